# baseline (speedup 1.0000x reference)
import functools

import jax
import jax.numpy as jnp
from jax import lax
from jax.experimental import pallas as pl
from jax.experimental.pallas import tpu as pltpu
from jax.experimental.pallas import tpu_sc as plsc

N = 50000
E = 800000
H = 32

NC = 2
NS = 16
CHUNK = 128
NPAD = 50048
ROWS_PER_SUB = NPAD // NS
ZROWS = 136


def _sc_edge_kernel(ps_hbm, pd_hbm, ep_hbm, s_hbm, r_hbm, out_hbm,
                    sidx0, ridx0, sidx1, ridx1,
                    ps0, pd0, ep0, ps1, pd1, ep1,
                    db0, agg_sh,
                    isem0, isem1, gsem0, gsem1):
    c = lax.axis_index("c")
    s = lax.axis_index("s")
    wid = s * NC + c
    sidx = (sidx0, sidx1)
    ridx = (ridx0, ridx1)
    psb = (ps0, ps1)
    pdb = (pd0, pd1)
    epb = (ep0, ep1)
    isem = (isem0, isem1)
    gsem = (gsem0, gsem1)

    num_chunks = E // CHUNK
    nb = num_chunks // (NC * NS)
    rem = num_chunks - nb * NC * NS

    def _zrow(i, _):
        db0[i, pl.ds(0, 16)] = jnp.zeros((16,), jnp.float32)
        db0[i, pl.ds(16, 16)] = jnp.zeros((16,), jnp.float32)
        return ()
    lax.fori_loop(0, ZROWS, _zrow, ())
    for t in range(ROWS_PER_SUB // ZROWS):
        pltpu.sync_copy(db0, agg_sh.at[pl.ds(s * ROWS_PER_SUB + t * ZROWS, ZROWS)])

    plsc.subcore_barrier()

    base = wid * nb

    def _issue_idx(t, slot):
        off = (base + t) * CHUNK
        pltpu.async_copy(s_hbm.at[pl.ds(off, CHUNK)], sidx[slot], isem[slot])
        pltpu.async_copy(r_hbm.at[pl.ds(off, CHUNK)], ridx[slot], isem[slot])

    def _wait_idx(slot):
        pltpu.make_async_copy(s_hbm.at[pl.ds(0, CHUNK)], sidx[slot], isem[slot]).wait()
        pltpu.make_async_copy(r_hbm.at[pl.ds(0, CHUNK)], ridx[slot], isem[slot]).wait()

    def _issue_gather(t, slot):
        off = (base + t) * CHUNK
        pltpu.async_copy(ps_hbm.at[sidx[slot]], psb[slot], gsem[slot])
        pltpu.async_copy(pd_hbm.at[ridx[slot]], pdb[slot], gsem[slot])
        pltpu.async_copy(ep_hbm.at[pl.ds(off // 4, CHUNK // 4)], epb[slot], gsem[slot])

    def _wait_gather(slot):
        pltpu.make_async_copy(ps_hbm.at[pl.ds(0, CHUNK)], psb[slot], gsem[slot]).wait()
        pltpu.make_async_copy(pd_hbm.at[pl.ds(0, CHUNK)], pdb[slot], gsem[slot]).wait()
        pltpu.make_async_copy(ep_hbm.at[pl.ds(0, CHUNK // 4)], epb[slot], gsem[slot]).wait()

    def _compute(slot):
        pv, dv, ev = psb[slot], pdb[slot], epb[slot]

        @plsc.parallel_loop(0, CHUNK // 4, 1, unroll=2)
        def _edge(m):
            for q in range(4):
                i = 4 * m + q
                for j in (0, 16):
                    sl = pl.ds(j, 16)
                    pv[i, sl] = jnp.maximum(
                        pv[i, sl] + dv[i, sl] + ev[m, pl.ds(32 * q + j, 16)], 0.0)

    def _step(t, slot):
        @pl.when(t + 1 < nb)
        def _():
            _wait_idx(1 - slot)
            _issue_gather(t + 1, 1 - slot)
        _wait_gather(slot)
        _compute(slot)
        pltpu.sync_copy(psb[slot], agg_sh.at[ridx[slot]], add=True)

        @pl.when(t + 2 < nb)
        def _():
            _issue_idx(t + 2, slot)

    _issue_idx(0, 0)
    _issue_idx(1, 1)
    _wait_idx(0)
    _issue_gather(0, 0)

    def _pair(g, _):
        _step(2 * g, 0)
        _step(2 * g + 1, 1)
        return ()
    lax.fori_loop(0, nb // 2, _pair, ())
    if nb % 2:
        _step(nb - 1, (nb - 1) % 2)

    @pl.when(wid < rem)
    def _tail():
        off = (nb * NC * NS + wid) * CHUNK
        pltpu.sync_copy(s_hbm.at[pl.ds(off, CHUNK)], sidx0)
        pltpu.sync_copy(r_hbm.at[pl.ds(off, CHUNK)], ridx0)
        pltpu.async_copy(ps_hbm.at[sidx0], ps0, gsem0)
        pltpu.async_copy(pd_hbm.at[ridx0], pd0, gsem0)
        pltpu.async_copy(ep_hbm.at[pl.ds(off // 4, CHUNK // 4)], ep0, gsem0)
        _wait_gather(0)
        _compute(0)
        pltpu.sync_copy(ps0, agg_sh.at[ridx0], add=True)

    plsc.subcore_barrier()

    for t in range(ROWS_PER_SUB // ZROWS):
        rb = s * ROWS_PER_SUB + t * ZROWS
        pltpu.sync_copy(agg_sh.at[pl.ds(rb, ZROWS)], db0)
        pltpu.sync_copy(db0, out_hbm.at[c].at[pl.ds(rb, ZROWS)])


@functools.cache
def _sc_edge_built():
    return pl.kernel(
        _sc_edge_kernel,
        mesh=plsc.VectorSubcoreMesh(core_axis_name="c", subcore_axis_name="s",
                                    num_cores=NC, num_subcores=NS),
        out_type=jax.ShapeDtypeStruct((NC, NPAD, H), jnp.float32),
        scratch_types=(
            [pltpu.VMEM((CHUNK,), jnp.int32)] * 4
            + [pltpu.VMEM((CHUNK, H), jnp.float32)] * 2
            + [pltpu.VMEM((CHUNK // 4, 4 * H), jnp.float32)]
            + [pltpu.VMEM((CHUNK, H), jnp.float32)] * 2
            + [pltpu.VMEM((CHUNK // 4, 4 * H), jnp.float32)]
            + [pltpu.VMEM((ZROWS, H), jnp.float32)]
            + [pltpu.VMEM_SHARED((NPAD, H), jnp.float32)]
            + [pltpu.SemaphoreType.DMA] * 4
        ),
        compiler_params=pltpu.CompilerParams(use_tc_tiling_on_sc=False),
    )


def _sc_edge(ps, pd, ep, s, r):
    return _sc_edge_built()(ps, pd, ep, s, r)


NBLK = 2000
EBLK = 6400


def _full(shape):
    return pl.BlockSpec(shape, lambda i: tuple(0 for _ in shape))


def _init_body(pq, w_in, b_in, wms, bm, wmd, v_out, h_out, ps_out, pd_out):
    h = jnp.dot(pq[...], w_in[...], preferred_element_type=jnp.float32) + b_in[...]
    h_out[...] = h
    v0 = jnp.concatenate([jnp.ones((NBLK, 1), jnp.float32),
                          jnp.zeros((NBLK, 1), jnp.float32)], axis=1)
    v_out[...] = v0
    vin = jnp.concatenate([v0, h], axis=1)
    ps_out[...] = jnp.dot(vin, wms[...], preferred_element_type=jnp.float32) + bm[...]
    pd_out[...] = jnp.dot(vin, wmd[...], preferred_element_type=jnp.float32)


def _tc_init(pq, w_in, b_in, wms, bm, wmd):
    grid = (N // NBLK,)
    row = lambda i: (i, 0)
    return pl.pallas_call(
        _init_body,
        grid=grid,
        in_specs=[
            pl.BlockSpec((NBLK, 2), row),
            _full((2, H)), _full((1, H)),
            _full((2 + H, H)), _full((1, H)), _full((2 + H, H)),
        ],
        out_specs=[pl.BlockSpec((NBLK, 2), row)] + [pl.BlockSpec((NBLK, H), row)] * 3,
        out_shape=[jax.ShapeDtypeStruct((N, 2), jnp.float32)]
        + [jax.ShapeDtypeStruct((N, H), jnp.float32)] * 3,
    )(pq, w_in, b_in, wms, bm, wmd)


EP_ROWS = 204800
EPBLK = 2048


def _eproj_body(ef4, wbig, out):
    out[...] = jnp.dot(ef4[...], wbig[...], preferred_element_type=jnp.float32)


def _tc_eproj(ef4, wbig):
    grid = (EP_ROWS // EPBLK,)
    return pl.pallas_call(
        _eproj_body,
        grid=grid,
        in_specs=[pl.BlockSpec((EPBLK, 16), lambda i: (i, 0)), _full((16, 4 * H))],
        out_specs=pl.BlockSpec((EPBLK, 4 * H), lambda i: (i, 0)),
        out_shape=jax.ShapeDtypeStruct((EP_ROWS, 4 * H), jnp.float32),
    )(ef4, wbig)


def _make_update(emit_next):
    def body(*refs):
        i = 0
        v = refs[i][...]; i += 1
        h = refs[i][...]; i += 1
        agg = refs[i][0] + refs[i + 1][0]; i += 2
        wu = refs[i][...]; i += 1
        bu = refs[i][...]; i += 1
        wdv = refs[i][...]; i += 1
        bdv = refs[i][...]; i += 1
        if emit_next:
            wms = refs[i][...]; i += 1
            bm = refs[i][...]; i += 1
            wmd = refs[i][...]; i += 1

        cat = jnp.concatenate([v, h, agg], axis=1)
        vout = jnp.dot(cat, wu, preferred_element_type=jnp.float32) + bu
        h2 = jnp.maximum(vout, 0.0)
        v2 = (v + jnp.dot(h2, wdv, preferred_element_type=jnp.float32)) + bdv

        refs[i][...] = v2; i += 1
        if emit_next:
            refs[i][...] = h2; i += 1
            vin2 = jnp.concatenate([v2, h2], axis=1)
            refs[i][...] = jnp.dot(vin2, wms, preferred_element_type=jnp.float32) + bm
            i += 1
            refs[i][...] = jnp.dot(vin2, wmd, preferred_element_type=jnp.float32)

    def call(v, h, agg01, weights):
        grid = (N // NBLK,)
        row = lambda i: (i, 0)
        in_specs = [pl.BlockSpec((NBLK, 2), row), pl.BlockSpec((NBLK, H), row),
                    pl.BlockSpec((1, NBLK, H), lambda i: (0, i, 0)),
                    pl.BlockSpec((1, NBLK, H), lambda i: (1, i, 0))]
        args = [v, h, agg01, agg01]
        for w in weights:
            in_specs.append(_full(w.shape)); args.append(w)
        out_specs = [pl.BlockSpec((NBLK, 2), row)]
        out_shape = [jax.ShapeDtypeStruct((N, 2), jnp.float32)]
        if emit_next:
            out_specs += [pl.BlockSpec((NBLK, H), row)] * 3
            out_shape += [jax.ShapeDtypeStruct((N, H), jnp.float32)] * 3
        return pl.pallas_call(
            body, grid=grid, in_specs=in_specs,
            out_specs=out_specs, out_shape=out_shape,
        )(*args)

    return call


_update_mid = _make_update(emit_next=True)
_update_last = _make_update(emit_next=False)


def kernel(P_Q_inj, senders, receivers, edge_features,
           W_in, b_in, W_msg, b_msg, W_upd, b_upd, W_dv, b_dv):
    r2 = lambda x: x.reshape(1, -1).astype(jnp.float32)

    wms = [W_msg[l, :2 + H, :] for l in range(3)]
    wmd = [W_msg[l, 2 + H:2 * (2 + H), :] for l in range(3)]
    wme = [W_msg[l, 2 * (2 + H):, :] for l in range(3)]

    v, h, ps, pd = _tc_init(P_Q_inj, W_in, r2(b_in), wms[0], r2(b_msg[0]), wmd[0])
    ef4 = jnp.pad(edge_features.reshape(E // 4, 16),
                  ((0, EP_ROWS - E // 4), (0, 0)))
    eye4 = jnp.eye(4, dtype=jnp.float32)
    eps = [_tc_eproj(ef4, jnp.kron(eye4, wme[l])) for l in range(3)]

    for l in range(2):
        agg01 = _sc_edge(ps, pd, eps[l], senders, receivers)
        v, h, ps, pd = _update_mid(
            v, h, agg01,
            [W_upd[l], r2(b_upd[l]), W_dv[l], r2(b_dv[l]),
             wms[l + 1], r2(b_msg[l + 1]), wmd[l + 1]])

    agg01 = _sc_edge(ps, pd, eps[2], senders, receivers)
    (v,) = _update_last(
        v, h, agg01, [W_upd[2], r2(b_upd[2]), W_dv[2], r2(b_dv[2])])
    return v


# speedup vs baseline: 1.0289x; 1.0289x over previous
import functools

import jax
import jax.numpy as jnp
from jax import lax
from jax.experimental import pallas as pl
from jax.experimental.pallas import tpu as pltpu
from jax.experimental.pallas import tpu_sc as plsc

N = 50000
E = 800000
H = 32

NC = 2
NS = 16
CHUNK = 128
NPAD = 50048
ROWS_PER_SUB = NPAD // NS
ZROWS = 136


def _sc_edge_kernel(ps_hbm, pd_hbm, ep_hbm, s_hbm, r_hbm, out_hbm,
                    sidx0, ridx0, sidx1, ridx1,
                    ps0, pd0, ep0, ps1, pd1, ep1,
                    db0, agg_sh,
                    isem0, isem1, gsem0, gsem1):
    c = lax.axis_index("c")
    s = lax.axis_index("s")
    wid = s * NC + c
    sidx = (sidx0, sidx1)
    ridx = (ridx0, ridx1)
    psb = (ps0, ps1)
    pdb = (pd0, pd1)
    epb = (ep0, ep1)
    isem = (isem0, isem1)
    gsem = (gsem0, gsem1)

    num_chunks = E // CHUNK
    nb = num_chunks // (NC * NS)
    rem = num_chunks - nb * NC * NS

    def _zrow(i, _):
        db0[i, pl.ds(0, 16)] = jnp.zeros((16,), jnp.float32)
        db0[i, pl.ds(16, 16)] = jnp.zeros((16,), jnp.float32)
        return ()
    lax.fori_loop(0, ZROWS, _zrow, ())
    for t in range(ROWS_PER_SUB // ZROWS):
        pltpu.sync_copy(db0, agg_sh.at[pl.ds(s * ROWS_PER_SUB + t * ZROWS, ZROWS)])

    plsc.subcore_barrier()

    base = wid * nb

    def _issue_idx(t, slot):
        off = (base + t) * CHUNK
        pltpu.async_copy(s_hbm.at[pl.ds(off, CHUNK)], sidx[slot], isem[slot])
        pltpu.async_copy(r_hbm.at[pl.ds(off, CHUNK)], ridx[slot], isem[slot])

    def _wait_idx(slot):
        pltpu.make_async_copy(s_hbm.at[pl.ds(0, CHUNK)], sidx[slot], isem[slot]).wait()
        pltpu.make_async_copy(r_hbm.at[pl.ds(0, CHUNK)], ridx[slot], isem[slot]).wait()

    def _issue_gather(t, slot):
        off = (base + t) * CHUNK
        pltpu.async_copy(ps_hbm.at[sidx[slot]], psb[slot], gsem[slot])
        pltpu.async_copy(pd_hbm.at[ridx[slot]], pdb[slot], gsem[slot])
        pltpu.async_copy(ep_hbm.at[pl.ds(off // 4, CHUNK // 4)], epb[slot], gsem[slot])

    def _wait_gather(slot):
        pltpu.make_async_copy(ps_hbm.at[pl.ds(0, CHUNK)], psb[slot], gsem[slot]).wait()
        pltpu.make_async_copy(pd_hbm.at[pl.ds(0, CHUNK)], pdb[slot], gsem[slot]).wait()
        pltpu.make_async_copy(ep_hbm.at[pl.ds(0, CHUNK // 4)], epb[slot], gsem[slot]).wait()

    def _compute(slot):
        pv, dv, ev = psb[slot], pdb[slot], epb[slot]

        @plsc.parallel_loop(0, CHUNK // 4, 1, unroll=2)
        def _edge(m):
            for q in range(4):
                i = 4 * m + q
                for j in (0, 16):
                    sl = pl.ds(j, 16)
                    pv[i, sl] = jnp.maximum(
                        pv[i, sl] + dv[i, sl] + ev[m, pl.ds(32 * q + j, 16)], 0.0)

    def _step(t, slot):
        @pl.when(t + 1 < nb)
        def _():
            _wait_idx(1 - slot)
            _issue_gather(t + 1, 1 - slot)
        _wait_gather(slot)
        _compute(slot)
        pltpu.sync_copy(psb[slot], agg_sh.at[ridx[slot]], add=True)

        @pl.when(t + 2 < nb)
        def _():
            _issue_idx(t + 2, slot)

    _issue_idx(0, 0)
    _issue_idx(1, 1)
    _wait_idx(0)
    _issue_gather(0, 0)

    def _pair(g, _):
        _step(2 * g, 0)
        _step(2 * g + 1, 1)
        return ()
    lax.fori_loop(0, nb // 2, _pair, ())
    if nb % 2:
        _step(nb - 1, (nb - 1) % 2)

    @pl.when(wid < rem)
    def _tail():
        off = (nb * NC * NS + wid) * CHUNK
        pltpu.sync_copy(s_hbm.at[pl.ds(off, CHUNK)], sidx0)
        pltpu.sync_copy(r_hbm.at[pl.ds(off, CHUNK)], ridx0)
        pltpu.async_copy(ps_hbm.at[sidx0], ps0, gsem0)
        pltpu.async_copy(pd_hbm.at[ridx0], pd0, gsem0)
        pltpu.async_copy(ep_hbm.at[pl.ds(off // 4, CHUNK // 4)], ep0, gsem0)
        _wait_gather(0)
        _compute(0)
        pltpu.sync_copy(ps0, agg_sh.at[ridx0], add=True)

    plsc.subcore_barrier()

    for t in range(ROWS_PER_SUB // ZROWS):
        rb = s * ROWS_PER_SUB + t * ZROWS
        pltpu.sync_copy(agg_sh.at[pl.ds(rb, ZROWS)], db0)
        pltpu.sync_copy(db0, out_hbm.at[c].at[pl.ds(rb, ZROWS)])


@functools.cache
def _sc_edge_built():
    return pl.kernel(
        _sc_edge_kernel,
        mesh=plsc.VectorSubcoreMesh(core_axis_name="c", subcore_axis_name="s",
                                    num_cores=NC, num_subcores=NS),
        out_type=jax.ShapeDtypeStruct((NC, NPAD, H), jnp.float32),
        scratch_types=(
            [pltpu.VMEM((CHUNK,), jnp.int32)] * 4
            + [pltpu.VMEM((CHUNK, H), jnp.float32)] * 2
            + [pltpu.VMEM((CHUNK // 4, 4 * H), jnp.float32)]
            + [pltpu.VMEM((CHUNK, H), jnp.float32)] * 2
            + [pltpu.VMEM((CHUNK // 4, 4 * H), jnp.float32)]
            + [pltpu.VMEM((ZROWS, H), jnp.float32)]
            + [pltpu.VMEM_SHARED((NPAD, H), jnp.float32)]
            + [pltpu.SemaphoreType.DMA] * 4
        ),
        compiler_params=pltpu.CompilerParams(use_tc_tiling_on_sc=False),
    )


def _sc_edge(ps, pd, ep, s, r):
    return _sc_edge_built()(ps, pd, ep, s, r)


NBLK = 2000
EBLK = 6400


def _full(shape):
    return pl.BlockSpec(shape, lambda i: tuple(0 for _ in shape))


def _init_body(pq, w_in, b_in, wms, bm, wmd, v_out, h_out, ps_out, pd_out):
    h = jnp.dot(pq[...], w_in[...], preferred_element_type=jnp.float32) + b_in[...]
    h_out[...] = h
    v0 = jnp.concatenate([jnp.ones((NBLK, 1), jnp.float32),
                          jnp.zeros((NBLK, 1), jnp.float32)], axis=1)
    v_out[...] = v0
    vin = jnp.concatenate([v0, h], axis=1)
    ps_out[...] = jnp.dot(vin, wms[...], preferred_element_type=jnp.float32) + bm[...]
    pd_out[...] = jnp.dot(vin, wmd[...], preferred_element_type=jnp.float32)


def _tc_init(pq, w_in, b_in, wms, bm, wmd):
    grid = (N // NBLK,)
    row = lambda i: (i, 0)
    return pl.pallas_call(
        _init_body,
        grid=grid,
        in_specs=[
            pl.BlockSpec((NBLK, 2), row),
            _full((2, H)), _full((1, H)),
            _full((2 + H, H)), _full((1, H)), _full((2 + H, H)),
        ],
        out_specs=[pl.BlockSpec((NBLK, 2), row)] + [pl.BlockSpec((NBLK, H), row)] * 3,
        out_shape=[jax.ShapeDtypeStruct((N, 2), jnp.float32)]
        + [jax.ShapeDtypeStruct((N, H), jnp.float32)] * 3,
    )(pq, w_in, b_in, wms, bm, wmd)


EP_ROWS = E // 4
EPBLK = 2000


def _eproj_body(ef4, wbig, out):
    out[...] = jnp.dot(ef4[...], wbig[...], preferred_element_type=jnp.float32)


def _tc_eproj(ef4, wbig):
    grid = (EP_ROWS // EPBLK,)
    return pl.pallas_call(
        _eproj_body,
        grid=grid,
        in_specs=[pl.BlockSpec((EPBLK, 16), lambda i: (i, 0)), _full((16, 4 * H))],
        out_specs=pl.BlockSpec((EPBLK, 4 * H), lambda i: (i, 0)),
        out_shape=jax.ShapeDtypeStruct((EP_ROWS, 4 * H), jnp.float32),
    )(ef4, wbig)


def _make_update(emit_next):
    def body(*refs):
        i = 0
        v = refs[i][...]; i += 1
        h = refs[i][...]; i += 1
        agg = refs[i][0] + refs[i + 1][0]; i += 2
        wu = refs[i][...]; i += 1
        bu = refs[i][...]; i += 1
        wdv = refs[i][...]; i += 1
        bdv = refs[i][...]; i += 1
        if emit_next:
            wms = refs[i][...]; i += 1
            bm = refs[i][...]; i += 1
            wmd = refs[i][...]; i += 1

        cat = jnp.concatenate([v, h, agg], axis=1)
        vout = jnp.dot(cat, wu, preferred_element_type=jnp.float32) + bu
        h2 = jnp.maximum(vout, 0.0)
        v2 = (v + jnp.dot(h2, wdv, preferred_element_type=jnp.float32)) + bdv

        refs[i][...] = v2; i += 1
        if emit_next:
            refs[i][...] = h2; i += 1
            vin2 = jnp.concatenate([v2, h2], axis=1)
            refs[i][...] = jnp.dot(vin2, wms, preferred_element_type=jnp.float32) + bm
            i += 1
            refs[i][...] = jnp.dot(vin2, wmd, preferred_element_type=jnp.float32)

    def call(v, h, agg01, weights):
        grid = (N // NBLK,)
        row = lambda i: (i, 0)
        in_specs = [pl.BlockSpec((NBLK, 2), row), pl.BlockSpec((NBLK, H), row),
                    pl.BlockSpec((1, NBLK, H), lambda i: (0, i, 0)),
                    pl.BlockSpec((1, NBLK, H), lambda i: (1, i, 0))]
        args = [v, h, agg01, agg01]
        for w in weights:
            in_specs.append(_full(w.shape)); args.append(w)
        out_specs = [pl.BlockSpec((NBLK, 2), row)]
        out_shape = [jax.ShapeDtypeStruct((N, 2), jnp.float32)]
        if emit_next:
            out_specs += [pl.BlockSpec((NBLK, H), row)] * 3
            out_shape += [jax.ShapeDtypeStruct((N, H), jnp.float32)] * 3
        return pl.pallas_call(
            body, grid=grid, in_specs=in_specs,
            out_specs=out_specs, out_shape=out_shape,
        )(*args)

    return call


_update_mid = _make_update(emit_next=True)
_update_last = _make_update(emit_next=False)


def kernel(P_Q_inj, senders, receivers, edge_features,
           W_in, b_in, W_msg, b_msg, W_upd, b_upd, W_dv, b_dv):
    r2 = lambda x: x.reshape(1, -1).astype(jnp.float32)

    wms = [W_msg[l, :2 + H, :] for l in range(3)]
    wmd = [W_msg[l, 2 + H:2 * (2 + H), :] for l in range(3)]
    wme = [W_msg[l, 2 * (2 + H):, :] for l in range(3)]

    v, h, ps, pd = _tc_init(P_Q_inj, W_in, r2(b_in), wms[0], r2(b_msg[0]), wmd[0])
    ef4 = edge_features.reshape(E // 4, 16)
    eye4 = jnp.eye(4, dtype=jnp.float32)
    eps = [_tc_eproj(ef4, jnp.kron(eye4, wme[l])) for l in range(3)]

    for l in range(2):
        agg01 = _sc_edge(ps, pd, eps[l], senders, receivers)
        v, h, ps, pd = _update_mid(
            v, h, agg01,
            [W_upd[l], r2(b_upd[l]), W_dv[l], r2(b_dv[l]),
             wms[l + 1], r2(b_msg[l + 1]), wmd[l + 1]])

    agg01 = _sc_edge(ps, pd, eps[2], senders, receivers)
    (v,) = _update_last(
        v, h, agg01, [W_upd[2], r2(b_upd[2]), W_dv[2], r2(b_dv[2])])
    return v


# speedup vs baseline: 1.3626x; 1.3243x over previous
import functools

import jax
import jax.numpy as jnp
from jax import lax
from jax.experimental import pallas as pl
from jax.experimental.pallas import tpu as pltpu
from jax.experimental.pallas import tpu_sc as plsc

N = 50000
E = 800000
H = 32

NC = 2
NS = 16
CHUNK = 128
NPAD = 50048
ROWS_PER_SUB = NPAD // NS
ZROWS = 136


def _sc_edge_kernel(ps_hbm, pd_hbm, ep_hbm, s_hbm, r_hbm, out_hbm,
                    sidx0, ridx0, sidx1, ridx1,
                    ps0, pd0, ep0, ps1, pd1, ep1,
                    db0, agg_sh,
                    isem0, isem1, gsem0, gsem1):
    c = lax.axis_index("c")
    s = lax.axis_index("s")
    wid = s * NC + c
    sidx = (sidx0, sidx1)
    ridx = (ridx0, ridx1)
    psb = (ps0, ps1)
    pdb = (pd0, pd1)
    epb = (ep0, ep1)
    isem = (isem0, isem1)
    gsem = (gsem0, gsem1)

    num_chunks = E // CHUNK
    nb = num_chunks // (NC * NS)
    rem = num_chunks - nb * NC * NS

    def _zrow(i, _):
        db0[i, pl.ds(0, 16)] = jnp.zeros((16,), jnp.float32)
        db0[i, pl.ds(16, 16)] = jnp.zeros((16,), jnp.float32)
        return ()
    lax.fori_loop(0, ZROWS, _zrow, ())
    for t in range(ROWS_PER_SUB // ZROWS):
        pltpu.sync_copy(db0, agg_sh.at[pl.ds(s * ROWS_PER_SUB + t * ZROWS, ZROWS)])

    plsc.subcore_barrier()

    base = wid * nb

    def _issue_idx(t, slot):
        off = (base + t) * CHUNK
        pltpu.async_copy(s_hbm.at[pl.ds(off, CHUNK)], sidx[slot], isem[slot])
        pltpu.async_copy(r_hbm.at[pl.ds(off, CHUNK)], ridx[slot], isem[slot])

    def _wait_idx(slot):
        pltpu.make_async_copy(s_hbm.at[pl.ds(0, CHUNK)], sidx[slot], isem[slot]).wait()
        pltpu.make_async_copy(r_hbm.at[pl.ds(0, CHUNK)], ridx[slot], isem[slot]).wait()

    def _issue_gather(t, slot):
        off = (base + t) * CHUNK
        pltpu.async_copy(ps_hbm.at[sidx[slot]], psb[slot], gsem[slot])
        pltpu.async_copy(pd_hbm.at[ridx[slot]], pdb[slot], gsem[slot])
        pltpu.async_copy(ep_hbm.at[pl.ds(off // 4, CHUNK // 4)], epb[slot], gsem[slot])

    def _wait_gather(slot):
        pltpu.make_async_copy(ps_hbm.at[pl.ds(0, CHUNK)], psb[slot], gsem[slot]).wait()
        pltpu.make_async_copy(pd_hbm.at[pl.ds(0, CHUNK)], pdb[slot], gsem[slot]).wait()
        pltpu.make_async_copy(ep_hbm.at[pl.ds(0, CHUNK // 4)], epb[slot], gsem[slot]).wait()

    def _compute(slot):
        pv, dv, ev = psb[slot], pdb[slot], epb[slot]

        @plsc.parallel_loop(0, CHUNK // 4, 1, unroll=2)
        def _edge(m):
            for q in range(4):
                i = 4 * m + q
                for j in (0, 16):
                    sl = pl.ds(j, 16)
                    pv[i, sl] = jnp.maximum(
                        pv[i, sl] + dv[i, sl] + ev[m, pl.ds(32 * q + j, 16)], 0.0)

    def _step(t, slot):
        @pl.when(t + 1 < nb)
        def _():
            _wait_idx(1 - slot)
            _issue_gather(t + 1, 1 - slot)
        _wait_gather(slot)
        _compute(slot)
        pltpu.sync_copy(psb[slot], agg_sh.at[ridx[slot]], add=True)

        @pl.when(t + 2 < nb)
        def _():
            _issue_idx(t + 2, slot)

    _issue_idx(0, 0)
    _issue_idx(1, 1)
    _wait_idx(0)
    _issue_gather(0, 0)

    def _pair(g, _):
        _step(2 * g, 0)
        _step(2 * g + 1, 1)
        return ()
    lax.fori_loop(0, nb // 2, _pair, ())
    if nb % 2:
        _step(nb - 1, (nb - 1) % 2)

    @pl.when(wid < rem)
    def _tail():
        off = (nb * NC * NS + wid) * CHUNK
        pltpu.sync_copy(s_hbm.at[pl.ds(off, CHUNK)], sidx0)
        pltpu.sync_copy(r_hbm.at[pl.ds(off, CHUNK)], ridx0)
        pltpu.async_copy(ps_hbm.at[sidx0], ps0, gsem0)
        pltpu.async_copy(pd_hbm.at[ridx0], pd0, gsem0)
        pltpu.async_copy(ep_hbm.at[pl.ds(off // 4, CHUNK // 4)], ep0, gsem0)
        _wait_gather(0)
        _compute(0)
        pltpu.sync_copy(ps0, agg_sh.at[ridx0], add=True)

    plsc.subcore_barrier()

    for t in range(ROWS_PER_SUB // ZROWS):
        rb = s * ROWS_PER_SUB + t * ZROWS
        pltpu.sync_copy(agg_sh.at[pl.ds(rb, ZROWS)], db0)
        pltpu.sync_copy(db0, out_hbm.at[c].at[pl.ds(rb, ZROWS)])


@functools.cache
def _sc_edge_built():
    return pl.kernel(
        _sc_edge_kernel,
        mesh=plsc.VectorSubcoreMesh(core_axis_name="c", subcore_axis_name="s",
                                    num_cores=NC, num_subcores=NS),
        out_type=jax.ShapeDtypeStruct((NC, NPAD, H), jnp.float32),
        scratch_types=(
            [pltpu.VMEM((CHUNK,), jnp.int32)] * 4
            + [pltpu.VMEM((CHUNK, H), jnp.float32)] * 2
            + [pltpu.VMEM((CHUNK // 4, 4 * H), jnp.float32)]
            + [pltpu.VMEM((CHUNK, H), jnp.float32)] * 2
            + [pltpu.VMEM((CHUNK // 4, 4 * H), jnp.float32)]
            + [pltpu.VMEM((ZROWS, H), jnp.float32)]
            + [pltpu.VMEM_SHARED((NPAD, H), jnp.float32)]
            + [pltpu.SemaphoreType.DMA] * 4
        ),
        compiler_params=pltpu.CompilerParams(use_tc_tiling_on_sc=False),
    )


def _sc_edge(ps, pd, ep, s, r):
    return _sc_edge_built()(ps, pd, ep, s, r)


NBLK = 2000
EBLK = 6400


def _full(shape):
    return pl.BlockSpec(shape, lambda i: tuple(0 for _ in shape))


def _init_body(pq, w_in, b_in, wms, bm, wmd, v_out, h_out, ps_out, pd_out):
    h = jnp.dot(pq[...], w_in[...], preferred_element_type=jnp.float32) + b_in[...]
    h_out[...] = h
    v0 = jnp.concatenate([jnp.ones((NBLK, 1), jnp.float32),
                          jnp.zeros((NBLK, 1), jnp.float32)], axis=1)
    v_out[...] = v0
    vin = jnp.concatenate([v0, h], axis=1)
    ps_out[...] = jnp.dot(vin, wms[...], preferred_element_type=jnp.float32) + bm[...]
    pd_out[...] = jnp.dot(vin, wmd[...], preferred_element_type=jnp.float32)


def _tc_init(pq, w_in, b_in, wms, bm, wmd):
    grid = (N // NBLK,)
    row = lambda i: (i, 0)
    return pl.pallas_call(
        _init_body,
        grid=grid,
        in_specs=[
            pl.BlockSpec((NBLK, 2), row),
            _full((2, H)), _full((1, H)),
            _full((2 + H, H)), _full((1, H)), _full((2 + H, H)),
        ],
        out_specs=[pl.BlockSpec((NBLK, 2), row)] + [pl.BlockSpec((NBLK, H), row)] * 3,
        out_shape=[jax.ShapeDtypeStruct((N, 2), jnp.float32)]
        + [jax.ShapeDtypeStruct((N, H), jnp.float32)] * 3,
    )(pq, w_in, b_in, wms, bm, wmd)


EP_ROWS = E // 4
EPBLK = 8000


def _eproj_body(ef, w, out):
    x = ef[...].reshape(EPBLK // 4, 4, 4)
    parts = [jnp.dot(x[:, q, :], w[...], preferred_element_type=jnp.float32)
             for q in range(4)]
    out[...] = jnp.concatenate(parts, axis=1)


def _tc_eproj(ef, w):
    grid = (E // EPBLK,)
    return pl.pallas_call(
        _eproj_body,
        grid=grid,
        in_specs=[pl.BlockSpec((EPBLK, 4), lambda i: (i, 0)), _full((4, H))],
        out_specs=pl.BlockSpec((EPBLK // 4, 4 * H), lambda i: (i, 0)),
        out_shape=jax.ShapeDtypeStruct((EP_ROWS, 4 * H), jnp.float32),
    )(ef, w)


def _make_update(emit_next):
    def body(*refs):
        i = 0
        v = refs[i][...]; i += 1
        h = refs[i][...]; i += 1
        agg = refs[i][0] + refs[i + 1][0]; i += 2
        wu = refs[i][...]; i += 1
        bu = refs[i][...]; i += 1
        wdv = refs[i][...]; i += 1
        bdv = refs[i][...]; i += 1
        if emit_next:
            wms = refs[i][...]; i += 1
            bm = refs[i][...]; i += 1
            wmd = refs[i][...]; i += 1

        cat = jnp.concatenate([v, h, agg], axis=1)
        vout = jnp.dot(cat, wu, preferred_element_type=jnp.float32) + bu
        h2 = jnp.maximum(vout, 0.0)
        v2 = (v + jnp.dot(h2, wdv, preferred_element_type=jnp.float32)) + bdv

        refs[i][...] = v2; i += 1
        if emit_next:
            refs[i][...] = h2; i += 1
            vin2 = jnp.concatenate([v2, h2], axis=1)
            refs[i][...] = jnp.dot(vin2, wms, preferred_element_type=jnp.float32) + bm
            i += 1
            refs[i][...] = jnp.dot(vin2, wmd, preferred_element_type=jnp.float32)

    def call(v, h, agg01, weights):
        grid = (N // NBLK,)
        row = lambda i: (i, 0)
        in_specs = [pl.BlockSpec((NBLK, 2), row), pl.BlockSpec((NBLK, H), row),
                    pl.BlockSpec((1, NBLK, H), lambda i: (0, i, 0)),
                    pl.BlockSpec((1, NBLK, H), lambda i: (1, i, 0))]
        args = [v, h, agg01, agg01]
        for w in weights:
            in_specs.append(_full(w.shape)); args.append(w)
        out_specs = [pl.BlockSpec((NBLK, 2), row)]
        out_shape = [jax.ShapeDtypeStruct((N, 2), jnp.float32)]
        if emit_next:
            out_specs += [pl.BlockSpec((NBLK, H), row)] * 3
            out_shape += [jax.ShapeDtypeStruct((N, H), jnp.float32)] * 3
        return pl.pallas_call(
            body, grid=grid, in_specs=in_specs,
            out_specs=out_specs, out_shape=out_shape,
        )(*args)

    return call


_update_mid = _make_update(emit_next=True)
_update_last = _make_update(emit_next=False)


def kernel(P_Q_inj, senders, receivers, edge_features,
           W_in, b_in, W_msg, b_msg, W_upd, b_upd, W_dv, b_dv):
    r2 = lambda x: x.reshape(1, -1).astype(jnp.float32)

    wms = [W_msg[l, :2 + H, :] for l in range(3)]
    wmd = [W_msg[l, 2 + H:2 * (2 + H), :] for l in range(3)]
    wme = [W_msg[l, 2 * (2 + H):, :] for l in range(3)]

    v, h, ps, pd = _tc_init(P_Q_inj, W_in, r2(b_in), wms[0], r2(b_msg[0]), wmd[0])
    eps = [_tc_eproj(edge_features, wme[l]) for l in range(3)]

    for l in range(2):
        agg01 = _sc_edge(ps, pd, eps[l], senders, receivers)
        v, h, ps, pd = _update_mid(
            v, h, agg01,
            [W_upd[l], r2(b_upd[l]), W_dv[l], r2(b_dv[l]),
             wms[l + 1], r2(b_msg[l + 1]), wmd[l + 1]])

    agg01 = _sc_edge(ps, pd, eps[2], senders, receivers)
    (v,) = _update_last(
        v, h, agg01, [W_upd[2], r2(b_upd[2]), W_dv[2], r2(b_dv[2])])
    return v


# speedup vs baseline: 1.3647x; 1.0015x over previous
import functools

import jax
import jax.numpy as jnp
from jax import lax
from jax.experimental import pallas as pl
from jax.experimental.pallas import tpu as pltpu
from jax.experimental.pallas import tpu_sc as plsc

N = 50000
E = 800000
H = 32

NC = 2
NS = 16
CHUNK = 128
NPAD = 50048
ROWS_PER_SUB = NPAD // NS
ZROWS = 136


def _sc_edge_kernel(ps_hbm, pd_hbm, ep_hbm, s_hbm, r_hbm, out_hbm,
                    sidx0, ridx0, sidx1, ridx1,
                    ps0, pd0, ep0, ps1, pd1, ep1,
                    db0, agg_sh,
                    isem0, isem1, gsem0, gsem1):
    c = lax.axis_index("c")
    s = lax.axis_index("s")
    wid = s * NC + c
    sidx = (sidx0, sidx1)
    ridx = (ridx0, ridx1)
    psb = (ps0, ps1)
    pdb = (pd0, pd1)
    epb = (ep0, ep1)
    isem = (isem0, isem1)
    gsem = (gsem0, gsem1)

    num_chunks = E // CHUNK
    nb = num_chunks // (NC * NS)
    rem = num_chunks - nb * NC * NS

    def _zrow(i, _):
        db0[i, pl.ds(0, 16)] = jnp.zeros((16,), jnp.float32)
        db0[i, pl.ds(16, 16)] = jnp.zeros((16,), jnp.float32)
        return ()
    lax.fori_loop(0, ZROWS, _zrow, ())

    def _zero(t, _):
        pltpu.sync_copy(db0, agg_sh.at[pl.ds(s * ROWS_PER_SUB + t * ZROWS, ZROWS)])
        return ()
    lax.fori_loop(0, ROWS_PER_SUB // ZROWS, _zero, ())

    plsc.subcore_barrier()

    base = wid * nb

    def _issue_idx(t, slot):
        off = (base + t) * CHUNK
        pltpu.async_copy(s_hbm.at[pl.ds(off, CHUNK)], sidx[slot], isem[slot])
        pltpu.async_copy(r_hbm.at[pl.ds(off, CHUNK)], ridx[slot], isem[slot])

    def _wait_idx(slot):
        pltpu.make_async_copy(s_hbm.at[pl.ds(0, CHUNK)], sidx[slot], isem[slot]).wait()
        pltpu.make_async_copy(r_hbm.at[pl.ds(0, CHUNK)], ridx[slot], isem[slot]).wait()

    def _issue_gather(t, slot):
        off = (base + t) * CHUNK
        pltpu.async_copy(ps_hbm.at[sidx[slot]], psb[slot], gsem[slot])
        pltpu.async_copy(pd_hbm.at[ridx[slot]], pdb[slot], gsem[slot])
        pltpu.async_copy(ep_hbm.at[pl.ds(off // 4, CHUNK // 4)], epb[slot], gsem[slot])

    def _wait_gather(slot):
        pltpu.make_async_copy(ps_hbm.at[pl.ds(0, CHUNK)], psb[slot], gsem[slot]).wait()
        pltpu.make_async_copy(pd_hbm.at[pl.ds(0, CHUNK)], pdb[slot], gsem[slot]).wait()
        pltpu.make_async_copy(ep_hbm.at[pl.ds(0, CHUNK // 4)], epb[slot], gsem[slot]).wait()

    def _compute(slot):
        pv, dv, ev = psb[slot], pdb[slot], epb[slot]

        @plsc.parallel_loop(0, CHUNK // 4, 1, unroll=2)
        def _edge(m):
            for q in range(4):
                i = 4 * m + q
                for j in (0, 16):
                    sl = pl.ds(j, 16)
                    pv[i, sl] = jnp.maximum(
                        pv[i, sl] + dv[i, sl] + ev[m, pl.ds(32 * q + j, 16)], 0.0)

    def _step(t, slot):
        @pl.when(t + 1 < nb)
        def _():
            _wait_idx(1 - slot)
            _issue_gather(t + 1, 1 - slot)
        _wait_gather(slot)
        _compute(slot)
        pltpu.sync_copy(psb[slot], agg_sh.at[ridx[slot]], add=True)

        @pl.when(t + 2 < nb)
        def _():
            _issue_idx(t + 2, slot)

    _issue_idx(0, 0)
    _issue_idx(1, 1)
    _wait_idx(0)
    _issue_gather(0, 0)

    def _pair(g, _):
        _step(2 * g, 0)
        _step(2 * g + 1, 1)
        return ()
    lax.fori_loop(0, nb // 2, _pair, ())
    if nb % 2:
        _step(nb - 1, (nb - 1) % 2)

    @pl.when(wid < rem)
    def _tail():
        off = (nb * NC * NS + wid) * CHUNK
        pltpu.sync_copy(s_hbm.at[pl.ds(off, CHUNK)], sidx0)
        pltpu.sync_copy(r_hbm.at[pl.ds(off, CHUNK)], ridx0)
        pltpu.async_copy(ps_hbm.at[sidx0], ps0, gsem0)
        pltpu.async_copy(pd_hbm.at[ridx0], pd0, gsem0)
        pltpu.async_copy(ep_hbm.at[pl.ds(off // 4, CHUNK // 4)], ep0, gsem0)
        _wait_gather(0)
        _compute(0)
        pltpu.sync_copy(ps0, agg_sh.at[ridx0], add=True)

    plsc.subcore_barrier()

    def _drain(t, _):
        rb = s * ROWS_PER_SUB + t * ZROWS
        pltpu.sync_copy(agg_sh.at[pl.ds(rb, ZROWS)], db0)
        pltpu.sync_copy(db0, out_hbm.at[c].at[pl.ds(rb, ZROWS)])
        return ()
    lax.fori_loop(0, ROWS_PER_SUB // ZROWS, _drain, ())


@functools.cache
def _sc_edge_built():
    return pl.kernel(
        _sc_edge_kernel,
        mesh=plsc.VectorSubcoreMesh(core_axis_name="c", subcore_axis_name="s",
                                    num_cores=NC, num_subcores=NS),
        out_type=jax.ShapeDtypeStruct((NC, NPAD, H), jnp.float32),
        scratch_types=(
            [pltpu.VMEM((CHUNK,), jnp.int32)] * 4
            + [pltpu.VMEM((CHUNK, H), jnp.float32)] * 2
            + [pltpu.VMEM((CHUNK // 4, 4 * H), jnp.float32)]
            + [pltpu.VMEM((CHUNK, H), jnp.float32)] * 2
            + [pltpu.VMEM((CHUNK // 4, 4 * H), jnp.float32)]
            + [pltpu.VMEM((ZROWS, H), jnp.float32)]
            + [pltpu.VMEM_SHARED((NPAD, H), jnp.float32)]
            + [pltpu.SemaphoreType.DMA] * 4
        ),
        compiler_params=pltpu.CompilerParams(use_tc_tiling_on_sc=False),
    )


def _sc_edge(ps, pd, ep, s, r):
    return _sc_edge_built()(ps, pd, ep, s, r)


NBLK = 2000
EBLK = 6400


def _full(shape):
    return pl.BlockSpec(shape, lambda i: tuple(0 for _ in shape))


def _init_body(pq, w_in, b_in, wms, bm, wmd, v_out, h_out, ps_out, pd_out):
    h = jnp.dot(pq[...], w_in[...], preferred_element_type=jnp.float32) + b_in[...]
    h_out[...] = h
    v0 = jnp.concatenate([jnp.ones((NBLK, 1), jnp.float32),
                          jnp.zeros((NBLK, 1), jnp.float32)], axis=1)
    v_out[...] = v0
    vin = jnp.concatenate([v0, h], axis=1)
    ps_out[...] = jnp.dot(vin, wms[...], preferred_element_type=jnp.float32) + bm[...]
    pd_out[...] = jnp.dot(vin, wmd[...], preferred_element_type=jnp.float32)


def _tc_init(pq, w_in, b_in, wms, bm, wmd):
    grid = (N // NBLK,)
    row = lambda i: (i, 0)
    return pl.pallas_call(
        _init_body,
        grid=grid,
        in_specs=[
            pl.BlockSpec((NBLK, 2), row),
            _full((2, H)), _full((1, H)),
            _full((2 + H, H)), _full((1, H)), _full((2 + H, H)),
        ],
        out_specs=[pl.BlockSpec((NBLK, 2), row)] + [pl.BlockSpec((NBLK, H), row)] * 3,
        out_shape=[jax.ShapeDtypeStruct((N, 2), jnp.float32)]
        + [jax.ShapeDtypeStruct((N, H), jnp.float32)] * 3,
    )(pq, w_in, b_in, wms, bm, wmd)


EP_ROWS = E // 4
EPBLK = 8000


def _eproj_body(ef, w, out):
    x = ef[...].reshape(EPBLK // 4, 4, 4)
    parts = [jnp.dot(x[:, q, :], w[...], preferred_element_type=jnp.float32)
             for q in range(4)]
    out[...] = jnp.concatenate(parts, axis=1)


def _tc_eproj(ef, w):
    grid = (E // EPBLK,)
    return pl.pallas_call(
        _eproj_body,
        grid=grid,
        in_specs=[pl.BlockSpec((EPBLK, 4), lambda i: (i, 0)), _full((4, H))],
        out_specs=pl.BlockSpec((EPBLK // 4, 4 * H), lambda i: (i, 0)),
        out_shape=jax.ShapeDtypeStruct((EP_ROWS, 4 * H), jnp.float32),
    )(ef, w)


def _make_update(emit_next):
    def body(*refs):
        i = 0
        v = refs[i][...]; i += 1
        h = refs[i][...]; i += 1
        agg = refs[i][0] + refs[i + 1][0]; i += 2
        wu = refs[i][...]; i += 1
        bu = refs[i][...]; i += 1
        wdv = refs[i][...]; i += 1
        bdv = refs[i][...]; i += 1
        if emit_next:
            wms = refs[i][...]; i += 1
            bm = refs[i][...]; i += 1
            wmd = refs[i][...]; i += 1

        cat = jnp.concatenate([v, h, agg], axis=1)
        vout = jnp.dot(cat, wu, preferred_element_type=jnp.float32) + bu
        h2 = jnp.maximum(vout, 0.0)
        v2 = (v + jnp.dot(h2, wdv, preferred_element_type=jnp.float32)) + bdv

        refs[i][...] = v2; i += 1
        if emit_next:
            refs[i][...] = h2; i += 1
            vin2 = jnp.concatenate([v2, h2], axis=1)
            refs[i][...] = jnp.dot(vin2, wms, preferred_element_type=jnp.float32) + bm
            i += 1
            refs[i][...] = jnp.dot(vin2, wmd, preferred_element_type=jnp.float32)

    def call(v, h, agg01, weights):
        grid = (N // NBLK,)
        row = lambda i: (i, 0)
        in_specs = [pl.BlockSpec((NBLK, 2), row), pl.BlockSpec((NBLK, H), row),
                    pl.BlockSpec((1, NBLK, H), lambda i: (0, i, 0)),
                    pl.BlockSpec((1, NBLK, H), lambda i: (1, i, 0))]
        args = [v, h, agg01, agg01]
        for w in weights:
            in_specs.append(_full(w.shape)); args.append(w)
        out_specs = [pl.BlockSpec((NBLK, 2), row)]
        out_shape = [jax.ShapeDtypeStruct((N, 2), jnp.float32)]
        if emit_next:
            out_specs += [pl.BlockSpec((NBLK, H), row)] * 3
            out_shape += [jax.ShapeDtypeStruct((N, H), jnp.float32)] * 3
        return pl.pallas_call(
            body, grid=grid, in_specs=in_specs,
            out_specs=out_specs, out_shape=out_shape,
        )(*args)

    return call


_update_mid = _make_update(emit_next=True)
_update_last = _make_update(emit_next=False)


def kernel(P_Q_inj, senders, receivers, edge_features,
           W_in, b_in, W_msg, b_msg, W_upd, b_upd, W_dv, b_dv):
    r2 = lambda x: x.reshape(1, -1).astype(jnp.float32)

    wms = [W_msg[l, :2 + H, :] for l in range(3)]
    wmd = [W_msg[l, 2 + H:2 * (2 + H), :] for l in range(3)]
    wme = [W_msg[l, 2 * (2 + H):, :] for l in range(3)]

    v, h, ps, pd = _tc_init(P_Q_inj, W_in, r2(b_in), wms[0], r2(b_msg[0]), wmd[0])
    eps = [_tc_eproj(edge_features, wme[l]) for l in range(3)]

    for l in range(2):
        agg01 = _sc_edge(ps, pd, eps[l], senders, receivers)
        v, h, ps, pd = _update_mid(
            v, h, agg01,
            [W_upd[l], r2(b_upd[l]), W_dv[l], r2(b_dv[l]),
             wms[l + 1], r2(b_msg[l + 1]), wmd[l + 1]])

    agg01 = _sc_edge(ps, pd, eps[2], senders, receivers)
    (v,) = _update_last(
        v, h, agg01, [W_upd[2], r2(b_upd[2]), W_dv[2], r2(b_dv[2])])
    return v


# speedup vs baseline: 1.3648x; 1.0001x over previous
import functools

import jax
import jax.numpy as jnp
from jax import lax
from jax.experimental import pallas as pl
from jax.experimental.pallas import tpu as pltpu
from jax.experimental.pallas import tpu_sc as plsc

N = 50000
E = 800000
H = 32

NC = 2
NS = 16
CHUNK = 128
NPAD = 50048
ROWS_PER_SUB = NPAD // NS
ZROWS = 136


def _sc_edge_kernel(ps_hbm, pd_hbm, ep_hbm, s_hbm, r_hbm, out_hbm,
                    sidx0, ridx0, sidx1, ridx1,
                    ps0, pd0, ep0, ps1, pd1, ep1,
                    db0, agg_sh,
                    isem0, isem1, gsem0, gsem1):
    c = lax.axis_index("c")
    s = lax.axis_index("s")
    wid = s * NC + c
    sidx = (sidx0, sidx1)
    ridx = (ridx0, ridx1)
    psb = (ps0, ps1)
    pdb = (pd0, pd1)
    epb = (ep0, ep1)
    isem = (isem0, isem1)
    gsem = (gsem0, gsem1)

    num_chunks = E // CHUNK
    nb = num_chunks // (NC * NS)
    rem = num_chunks - nb * NC * NS

    def _zrow(i, _):
        db0[i, pl.ds(0, 16)] = jnp.zeros((16,), jnp.float32)
        db0[i, pl.ds(16, 16)] = jnp.zeros((16,), jnp.float32)
        return ()
    lax.fori_loop(0, ZROWS, _zrow, ())

    def _zero(t, _):
        pltpu.sync_copy(db0, agg_sh.at[pl.ds(s * ROWS_PER_SUB + t * ZROWS, ZROWS)])
        return ()
    lax.fori_loop(0, ROWS_PER_SUB // ZROWS, _zero, ())

    plsc.subcore_barrier()

    base = wid * nb

    def _issue_idx(t, slot):
        off = (base + t) * CHUNK
        pltpu.async_copy(s_hbm.at[pl.ds(off, CHUNK)], sidx[slot], isem[slot])
        pltpu.async_copy(r_hbm.at[pl.ds(off, CHUNK)], ridx[slot], isem[slot])

    def _wait_idx(slot):
        pltpu.make_async_copy(s_hbm.at[pl.ds(0, CHUNK)], sidx[slot], isem[slot]).wait()
        pltpu.make_async_copy(r_hbm.at[pl.ds(0, CHUNK)], ridx[slot], isem[slot]).wait()

    def _issue_gather(t, slot):
        off = (base + t) * CHUNK
        pltpu.async_copy(ps_hbm.at[sidx[slot]], psb[slot], gsem[slot])
        pltpu.async_copy(pd_hbm.at[ridx[slot]], pdb[slot], gsem[slot])
        pltpu.async_copy(ep_hbm.at[pl.ds(off // 4, CHUNK // 4)], epb[slot], gsem[slot])

    def _wait_gather(slot):
        pltpu.make_async_copy(ps_hbm.at[pl.ds(0, CHUNK)], psb[slot], gsem[slot]).wait()
        pltpu.make_async_copy(pd_hbm.at[pl.ds(0, CHUNK)], pdb[slot], gsem[slot]).wait()
        pltpu.make_async_copy(ep_hbm.at[pl.ds(0, CHUNK // 4)], epb[slot], gsem[slot]).wait()

    def _compute(slot):
        pv, dv, ev = psb[slot], pdb[slot], epb[slot]

        @plsc.parallel_loop(0, CHUNK // 4, 1, unroll=2)
        def _edge(m):
            for q in range(4):
                i = 4 * m + q
                for j in (0, 16):
                    sl = pl.ds(j, 16)
                    pv[i, sl] = jnp.maximum(
                        pv[i, sl] + dv[i, sl] + ev[m, pl.ds(32 * q + j, 16)], 0.0)

    def _step(t, slot):
        @pl.when(t + 1 < nb)
        def _():
            _wait_idx(1 - slot)
            _issue_gather(t + 1, 1 - slot)
        _wait_gather(slot)
        _compute(slot)
        pltpu.sync_copy(psb[slot], agg_sh.at[ridx[slot]], add=True)

        @pl.when(t + 2 < nb)
        def _():
            _issue_idx(t + 2, slot)

    _issue_idx(0, 0)
    _issue_idx(1, 1)
    _wait_idx(0)
    _issue_gather(0, 0)

    def _pair(g, _):
        _step(2 * g, 0)
        _step(2 * g + 1, 1)
        return ()
    lax.fori_loop(0, nb // 2, _pair, ())
    if nb % 2:
        _step(nb - 1, (nb - 1) % 2)

    @pl.when(wid < rem)
    def _tail():
        off = (nb * NC * NS + wid) * CHUNK
        pltpu.sync_copy(s_hbm.at[pl.ds(off, CHUNK)], sidx0)
        pltpu.sync_copy(r_hbm.at[pl.ds(off, CHUNK)], ridx0)
        pltpu.async_copy(ps_hbm.at[sidx0], ps0, gsem0)
        pltpu.async_copy(pd_hbm.at[ridx0], pd0, gsem0)
        pltpu.async_copy(ep_hbm.at[pl.ds(off // 4, CHUNK // 4)], ep0, gsem0)
        _wait_gather(0)
        _compute(0)
        pltpu.sync_copy(ps0, agg_sh.at[ridx0], add=True)

    plsc.subcore_barrier()

    def _drain(t, _):
        rb = s * ROWS_PER_SUB + t * ZROWS
        pltpu.sync_copy(agg_sh.at[pl.ds(rb, ZROWS)], db0)
        pltpu.sync_copy(db0, out_hbm.at[c].at[pl.ds(rb, ZROWS)])
        return ()
    lax.fori_loop(0, ROWS_PER_SUB // ZROWS, _drain, ())


@functools.cache
def _sc_edge_built():
    return pl.kernel(
        _sc_edge_kernel,
        mesh=plsc.VectorSubcoreMesh(core_axis_name="c", subcore_axis_name="s",
                                    num_cores=NC, num_subcores=NS),
        out_type=jax.ShapeDtypeStruct((NC, NPAD, H), jnp.float32),
        scratch_types=(
            [pltpu.VMEM((CHUNK,), jnp.int32)] * 4
            + [pltpu.VMEM((CHUNK, H), jnp.float32)] * 2
            + [pltpu.VMEM((CHUNK // 4, 4 * H), jnp.float32)]
            + [pltpu.VMEM((CHUNK, H), jnp.float32)] * 2
            + [pltpu.VMEM((CHUNK // 4, 4 * H), jnp.float32)]
            + [pltpu.VMEM((ZROWS, H), jnp.float32)]
            + [pltpu.VMEM_SHARED((NPAD, H), jnp.float32)]
            + [pltpu.SemaphoreType.DMA] * 4
        ),
        compiler_params=pltpu.CompilerParams(use_tc_tiling_on_sc=False),
    )


def _sc_edge(ps, pd, ep, s, r):
    return _sc_edge_built()(ps, pd, ep, s, r)


NBLK = 2000


def _full(shape):
    return pl.BlockSpec(shape, lambda i: tuple(0 for _ in shape))


def _init_body(pq, w_in, b_in, wms, bm, wmd, v_out, h_out, ps_out, pd_out):
    h = jnp.dot(pq[...], w_in[...], preferred_element_type=jnp.float32) + b_in[...]
    h_out[...] = h
    v0 = jnp.concatenate([jnp.ones((NBLK, 1), jnp.float32),
                          jnp.zeros((NBLK, 1), jnp.float32)], axis=1)
    v_out[...] = v0
    vin = jnp.concatenate([v0, h], axis=1)
    ps_out[...] = jnp.dot(vin, wms[...], preferred_element_type=jnp.float32) + bm[...]
    pd_out[...] = jnp.dot(vin, wmd[...], preferred_element_type=jnp.float32)


def _tc_init(pq, w_in, b_in, wms, bm, wmd):
    grid = (N // NBLK,)
    row = lambda i: (i, 0)
    return pl.pallas_call(
        _init_body,
        grid=grid,
        in_specs=[
            pl.BlockSpec((NBLK, 2), row),
            _full((2, H)), _full((1, H)),
            _full((2 + H, H)), _full((1, H)), _full((2 + H, H)),
        ],
        out_specs=[pl.BlockSpec((NBLK, 2), row)] + [pl.BlockSpec((NBLK, H), row)] * 3,
        out_shape=[jax.ShapeDtypeStruct((N, 2), jnp.float32)]
        + [jax.ShapeDtypeStruct((N, H), jnp.float32)] * 3,
    )(pq, w_in, b_in, wms, bm, wmd)


EP_ROWS = E // 4
EPBLK = 8000


def _eproj_body(ef, w, out):
    x = ef[...].reshape(EPBLK // 4, 4, 4)
    parts = [jnp.dot(x[:, q, :], w[...], preferred_element_type=jnp.float32)
             for q in range(4)]
    out[...] = jnp.concatenate(parts, axis=1)


def _tc_eproj(ef, w):
    grid = (E // EPBLK,)
    return pl.pallas_call(
        _eproj_body,
        grid=grid,
        in_specs=[pl.BlockSpec((EPBLK, 4), lambda i: (i, 0)), _full((4, H))],
        out_specs=pl.BlockSpec((EPBLK // 4, 4 * H), lambda i: (i, 0)),
        out_shape=jax.ShapeDtypeStruct((EP_ROWS, 4 * H), jnp.float32),
    )(ef, w)


def _make_update(emit_next):
    def body(*refs):
        i = 0
        v = refs[i][...]; i += 1
        h = refs[i][...]; i += 1
        agg = refs[i][0] + refs[i + 1][0]; i += 2
        wu = refs[i][...]; i += 1
        bu = refs[i][...]; i += 1
        wdv = refs[i][...]; i += 1
        bdv = refs[i][...]; i += 1
        if emit_next:
            wms = refs[i][...]; i += 1
            bm = refs[i][...]; i += 1
            wmd = refs[i][...]; i += 1

        cat = jnp.concatenate([v, h, agg], axis=1)
        vout = jnp.dot(cat, wu, preferred_element_type=jnp.float32) + bu
        h2 = jnp.maximum(vout, 0.0)
        v2 = (v + jnp.dot(h2, wdv, preferred_element_type=jnp.float32)) + bdv

        refs[i][...] = v2; i += 1
        if emit_next:
            refs[i][...] = h2; i += 1
            vin2 = jnp.concatenate([v2, h2], axis=1)
            refs[i][...] = jnp.dot(vin2, wms, preferred_element_type=jnp.float32) + bm
            i += 1
            refs[i][...] = jnp.dot(vin2, wmd, preferred_element_type=jnp.float32)

    def call(v, h, agg01, weights):
        grid = (N // NBLK,)
        row = lambda i: (i, 0)
        in_specs = [pl.BlockSpec((NBLK, 2), row), pl.BlockSpec((NBLK, H), row),
                    pl.BlockSpec((1, NBLK, H), lambda i: (0, i, 0)),
                    pl.BlockSpec((1, NBLK, H), lambda i: (1, i, 0))]
        args = [v, h, agg01, agg01]
        for w in weights:
            in_specs.append(_full(w.shape)); args.append(w)
        out_specs = [pl.BlockSpec((NBLK, 2), row)]
        out_shape = [jax.ShapeDtypeStruct((N, 2), jnp.float32)]
        if emit_next:
            out_specs += [pl.BlockSpec((NBLK, H), row)] * 3
            out_shape += [jax.ShapeDtypeStruct((N, H), jnp.float32)] * 3
        return pl.pallas_call(
            body, grid=grid, in_specs=in_specs,
            out_specs=out_specs, out_shape=out_shape,
        )(*args)

    return call


_update_mid = _make_update(emit_next=True)
_update_last = _make_update(emit_next=False)


def kernel(P_Q_inj, senders, receivers, edge_features,
           W_in, b_in, W_msg, b_msg, W_upd, b_upd, W_dv, b_dv):
    r2 = lambda x: x.reshape(1, -1).astype(jnp.float32)

    wms = [W_msg[l, :2 + H, :] for l in range(3)]
    wmd = [W_msg[l, 2 + H:2 * (2 + H), :] for l in range(3)]
    wme = [W_msg[l, 2 * (2 + H):, :] for l in range(3)]

    v, h, ps, pd = _tc_init(P_Q_inj, W_in, r2(b_in), wms[0], r2(b_msg[0]), wmd[0])
    eps = [_tc_eproj(edge_features, wme[l]) for l in range(3)]

    for l in range(2):
        agg01 = _sc_edge(ps, pd, eps[l], senders, receivers)
        v, h, ps, pd = _update_mid(
            v, h, agg01,
            [W_upd[l], r2(b_upd[l]), W_dv[l], r2(b_dv[l]),
             wms[l + 1], r2(b_msg[l + 1]), wmd[l + 1]])

    agg01 = _sc_edge(ps, pd, eps[2], senders, receivers)
    (v,) = _update_last(
        v, h, agg01, [W_upd[2], r2(b_upd[2]), W_dv[2], r2(b_dv[2])])
    return v
